# initial kernel scaffold (unmeasured)
import jax
import jax.numpy as jnp
from jax import lax
from jax.experimental import pallas as pl
from jax.experimental.pallas import tpu as pltpu

N_DEV = 4


def _ag_body(x_ref, xg_ref, copy_sem, send_sems, recv_sems):
    my = lax.axis_index("i")
    right = lax.rem(my + 1, N_DEV)
    m_per = x_ref.shape[0]

    cp = pltpu.make_async_copy(
        x_ref, xg_ref.at[pl.ds(my * m_per, m_per), :], copy_sem
    )
    cp.start()
    cp.wait()

    for h in range(N_DEV - 1):
        src = lax.rem(my - h + N_DEV, N_DEV)
        rdma = pltpu.make_async_remote_copy(
            src_ref=xg_ref.at[pl.ds(src * m_per, m_per), :],
            dst_ref=xg_ref.at[pl.ds(src * m_per, m_per), :],
            send_sem=send_sems.at[h],
            recv_sem=recv_sems.at[h],
            device_id=(right,),
            device_id_type=pl.DeviceIdType.MESH,
        )
        rdma.start()
        rdma.wait()


def kernel(x, w_mat):
    m_per, k = x.shape
    xg = pl.pallas_call(
        _ag_body,
        out_shape=jax.ShapeDtypeStruct((N_DEV * m_per, k), x.dtype),
        in_specs=[pl.BlockSpec(memory_space=pltpu.ANY)],
        out_specs=pl.BlockSpec(memory_space=pltpu.ANY),
        scratch_shapes=[
            pltpu.SemaphoreType.DMA,
            pltpu.SemaphoreType.DMA((N_DEV - 1,)),
            pltpu.SemaphoreType.DMA((N_DEV - 1,)),
        ],
        compiler_params=pltpu.CompilerParams(collective_id=0),
    )(x)
    return jnp.dot(xg, w_mat, preferred_element_type=jnp.float32)


# baseline (device time: 4398834 ns/iter reference)
import jax
import jax.numpy as jnp
from jax import lax
from jax.experimental import pallas as pl
from jax.experimental.pallas import tpu as pltpu

N_DEV = 4


def _ag_body(x_ref, xg_ref, copy_sem, send_sems, recv_sems):
    my = lax.axis_index("i")
    right = lax.rem(my + 1, N_DEV)
    left = lax.rem(my - 1 + N_DEV, N_DEV)
    m_per = x_ref.shape[0]

    barrier_sem = pltpu.get_barrier_semaphore()
    for nbr in (left, right):
        pl.semaphore_signal(
            barrier_sem, inc=1,
            device_id=(nbr,), device_id_type=pl.DeviceIdType.MESH,
        )
    pl.semaphore_wait(barrier_sem, 2)

    cp = pltpu.make_async_copy(
        x_ref, xg_ref.at[pl.ds(my * m_per, m_per), :], copy_sem
    )
    cp.start()
    cp.wait()

    for h in range(N_DEV - 1):
        src = lax.rem(my - h + N_DEV, N_DEV)
        rdma = pltpu.make_async_remote_copy(
            src_ref=xg_ref.at[pl.ds(src * m_per, m_per), :],
            dst_ref=xg_ref.at[pl.ds(src * m_per, m_per), :],
            send_sem=send_sems.at[h],
            recv_sem=recv_sems.at[h],
            device_id=(right,),
            device_id_type=pl.DeviceIdType.MESH,
        )
        rdma.start()
        rdma.wait()


def kernel(x, w_mat):
    m_per, k = x.shape
    xg = pl.pallas_call(
        _ag_body,
        out_shape=jax.ShapeDtypeStruct((N_DEV * m_per, k), x.dtype),
        in_specs=[pl.BlockSpec(memory_space=pl.ANY)],
        out_specs=pl.BlockSpec(memory_space=pl.ANY),
        scratch_shapes=[
            pltpu.SemaphoreType.DMA,
            pltpu.SemaphoreType.DMA((N_DEV - 1,)),
            pltpu.SemaphoreType.DMA((N_DEV - 1,)),
        ],
        compiler_params=pltpu.CompilerParams(collective_id=0),
    )(x)
    return jnp.dot(xg, w_mat, preferred_element_type=jnp.float32)


# device time: 873206 ns/iter; 5.0376x vs baseline; 5.0376x over previous
import jax
import jax.numpy as jnp
from jax import lax
from jax.experimental import pallas as pl
from jax.experimental.pallas import tpu as pltpu

N_DEV = 4
M_PER = 2048
K = 8192
KH = K // 2
N_PER = 1024
PIECE = 256
NP = M_PER // PIECE
NP2 = NP // 2


def _body(x_ref, w_ref, y_ref, wg_ref, wbuf, xbuf, ybuf,
          w_sem, xin_sems, yloc_sems, ysend_sems, yrecv_sems,
          cw_send, cw_recv, ccw_send, ccw_recv):
    me = lax.axis_index("i")
    right = lax.rem(me + 1, N_DEV)
    left = lax.rem(me + N_DEV - 1, N_DEV)
    diag = lax.rem(me + 2, N_DEV)

    barrier_sem = pltpu.get_barrier_semaphore()
    for nbr in (left, right, diag):
        pl.semaphore_signal(
            barrier_sem, inc=1,
            device_id=(nbr,), device_id_type=pl.DeviceIdType.MESH,
        )
    pl.semaphore_wait(barrier_sem, 3)

    def ring(h, to_right):
        if to_right:
            dev, s_sems, r_sems = right, cw_send, cw_recv
            if h == 0:
                src, chunk, off = w_ref.at[pl.ds(0, KH), :], me, 0
            elif h == 1:
                src, chunk, off = w_ref.at[pl.ds(KH, KH), :], me, KH
            else:
                o = left
                src, chunk, off = wg_ref.at[o, pl.ds(0, KH), :], o, 0
        else:
            dev, s_sems, r_sems = left, ccw_send, ccw_recv
            if h == 0:
                src, chunk, off = w_ref.at[pl.ds(KH, KH), :], me, KH
            elif h == 1:
                src, chunk, off = w_ref.at[pl.ds(0, KH), :], me, 0
            else:
                o = right
                src, chunk, off = wg_ref.at[o, pl.ds(KH, KH), :], o, KH
        return pltpu.make_async_remote_copy(
            src_ref=src,
            dst_ref=wg_ref.at[chunk, pl.ds(off, KH), :],
            send_sem=s_sems.at[h],
            recv_sem=r_sems.at[h],
            device_id=(dev,),
            device_id_type=pl.DeviceIdType.MESH,
        )

    def y_send_desc(slot, row0, target):
        return pltpu.make_async_remote_copy(
            src_ref=ybuf.at[slot],
            dst_ref=y_ref.at[pl.ds(row0, PIECE), :],
            send_sem=ysend_sems.at[slot],
            recv_sem=yrecv_sems.at[me],
            device_id=(target,),
            device_id_type=pl.DeviceIdType.MESH,
        )

    def y_loc_desc(slot, row0):
        return pltpu.make_async_copy(
            ybuf.at[slot], y_ref.at[pl.ds(row0, PIECE), :], yloc_sems.at[slot]
        )

    def x_load_desc(slot, p):
        return pltpu.make_async_copy(
            x_ref.at[pl.ds(p * PIECE, PIECE), :], xbuf.at[slot], xin_sems.at[slot]
        )

    def gemm_chunk(w_src, target, remote):
        wld = pltpu.make_async_copy(w_src, wbuf, w_sem)
        wld.start()
        for slot in (0, 1):
            x_load_desc(slot, slot).start()
        wld.wait()

        def wait_y_slot(slot):
            if remote:
                y_send_desc(slot, 0, target).wait_send()
            else:
                y_loc_desc(slot, 0).wait()

        def piece_step(p2, carry):
            for slot in (0, 1):
                x_load_desc(slot, 0).wait()

                @pl.when(p2 > 0)
                def _():
                    wait_y_slot(slot)

                ybuf[slot] = jnp.dot(
                    xbuf[slot], wbuf[...], preferred_element_type=jnp.float32
                )

                @pl.when(p2 < NP2 - 1)
                def _():
                    x_load_desc(slot, 2 * p2 + 2 + slot).start()

                row0 = me * M_PER + (2 * p2 + slot) * PIECE
                if remote:
                    y_send_desc(slot, row0, target).start()
                else:
                    y_loc_desc(slot, row0).start()
            return carry

        lax.fori_loop(0, NP2, piece_step, 0)
        for slot in (0, 1):
            wait_y_slot(slot)

    hops = {}
    for h in (0, 1):
        for tr in (True, False):
            hops[(h, tr)] = ring(h, tr)
            hops[(h, tr)].start()

    gemm_chunk(w_ref, me, False)

    hops[(0, True)].wait_recv()
    hops[(0, False)].wait_recv()
    for tr in (True, False):
        hops[(2, tr)] = ring(2, tr)
        hops[(2, tr)].start()

    hops[(1, True)].wait_recv()
    hops[(1, False)].wait_recv()
    gemm_chunk(wg_ref.at[left], left, True)
    gemm_chunk(wg_ref.at[right], right, True)

    hops[(2, True)].wait_recv()
    hops[(2, False)].wait_recv()
    gemm_chunk(wg_ref.at[diag], diag, True)

    for d in hops.values():
        d.wait_send()

    for jj in (1, 2, 3):
        s = lax.rem(me + jj, N_DEV)
        for p in range(NP):
            pltpu.make_async_remote_copy(
                src_ref=ybuf.at[0],
                dst_ref=y_ref.at[pl.ds(s * M_PER + p * PIECE, PIECE), :],
                send_sem=ysend_sems.at[0],
                recv_sem=yrecv_sems.at[s],
                device_id=(s,),
                device_id_type=pl.DeviceIdType.MESH,
            ).wait_recv()


def kernel(x, w_mat):
    y, _wg = pl.pallas_call(
        _body,
        out_shape=(
            jax.ShapeDtypeStruct((N_DEV * M_PER, N_PER), jnp.float32),
            jax.ShapeDtypeStruct((N_DEV, K, N_PER), jnp.float32),
        ),
        in_specs=[
            pl.BlockSpec(memory_space=pl.ANY),
            pl.BlockSpec(memory_space=pl.ANY),
        ],
        out_specs=(
            pl.BlockSpec(memory_space=pl.ANY),
            pl.BlockSpec(memory_space=pl.ANY),
        ),
        scratch_shapes=[
            pltpu.VMEM((K, N_PER), jnp.float32),
            pltpu.VMEM((2, PIECE, K), jnp.float32),
            pltpu.VMEM((2, PIECE, N_PER), jnp.float32),
            pltpu.SemaphoreType.DMA,
            pltpu.SemaphoreType.DMA((2,)),
            pltpu.SemaphoreType.DMA((2,)),
            pltpu.SemaphoreType.DMA((2,)),
            pltpu.SemaphoreType.DMA((N_DEV,)),
            pltpu.SemaphoreType.DMA((3,)),
            pltpu.SemaphoreType.DMA((3,)),
            pltpu.SemaphoreType.DMA((3,)),
            pltpu.SemaphoreType.DMA((3,)),
        ],
        compiler_params=pltpu.CompilerParams(
            collective_id=0,
            vmem_limit_bytes=60 * 1024 * 1024,
        ),
    )(x, w_mat)
    return y
